# baseline (speedup 1.0000x reference)
import functools

import jax
import jax.numpy as jnp
from jax import lax
from jax.experimental import pallas as pl
from jax.experimental.pallas import tpu as pltpu
from jax.experimental.pallas import tpu_sc as plsc

L = 16
NC = 2
NS = 16
NW = NC * NS
BLK = 80
UB = BLK // L


def _mesh():
    return plsc.VectorSubcoreMesh(core_axis_name="c", subcore_axis_name="s")


def _wid():
    return lax.axis_index("s") * NC + lax.axis_index("c")


IBLK = 128


def _insert_body(n_nodes, chunk, gen, i_hbm, j_hbm, table_ref,
                 ibuf, jbuf, kb, ones, sem):
    nfull = chunk // IBLK
    tail = chunk - nfull * IBLK
    nrows = nfull + (1 if tail else 0)
    pad_key = n_nodes * n_nodes
    w = _wid()
    base = w * chunk
    pltpu.sync_copy(i_hbm.at[pl.ds(base, chunk)], ibuf)
    pltpu.sync_copy(j_hbm.at[pl.ds(base, chunk)], jbuf)
    for u in range(IBLK // L):
        ones[pl.ds(u * L, L)] = jnp.full((L,), gen, jnp.int32)

    @pl.loop(0, nfull)
    def _keys(b):
        for u in range(IBLK // L):
            s = b * IBLK + u * L
            iv = ibuf[pl.ds(s, L)]
            jv = jbuf[pl.ds(s, L)]
            kb[b, pl.ds(u * L, L)] = iv * n_nodes + jv

    if tail:
        assert tail % L == 0
        for u in range(IBLK // L):
            s = nfull * IBLK + u * L
            if u * L < tail:
                iv = ibuf[pl.ds(s, L)]
                jv = jbuf[pl.ds(s, L)]
                kb[nfull, pl.ds(u * L, L)] = iv * n_nodes + jv
            else:
                kb[nfull, pl.ds(u * L, L)] = jnp.full((L,), pad_key, jnp.int32)

    @pl.loop(0, nrows)
    def _fire(b):
        pltpu.async_copy(ones, table_ref.at[kb.at[b]], sem)

    @pl.loop(0, nrows)
    def _drain(b):
        pltpu.make_async_copy(ones, table_ref.at[kb.at[b]], sem).wait()


def _query_body(n_nodes, chunk, gen, sign,
                i_hbm, j_hbm, k0_hbm, kn1_hbm, kn2_hbm, kn3_hbm,
                z_hbm, table_ref, out_hbm,
                ibuf, jbuf, kcur, knbuf, kb, cnt, zi, zj, zk, acc31, vres,
                sem):
    nb = chunk // BLK
    w = _wid()
    base = w * chunk
    pltpu.sync_copy(i_hbm.at[pl.ds(base, chunk)], ibuf)
    pltpu.sync_copy(k0_hbm.at[pl.ds(base, chunk)], kcur)

    nq = 5
    qb = nb // nq
    for t, kn_hbm in enumerate((kn1_hbm, kn2_hbm, kn3_hbm)):
        pltpu.sync_copy(kn_hbm.at[pl.ds(base, chunk)], knbuf)
        for q in range(nq):
            qoff = q * qb * BLK

            @pl.loop(0, qb)
            def _keys(r):
                for u in range(UB):
                    s = qoff + r * BLK + u * L
                    iv = ibuf[pl.ds(s, L)]
                    kv = kcur[pl.ds(s, L)]
                    kb[r, pl.ds(u * L, L)] = iv * n_nodes + kv

            @pl.loop(0, qb)
            def _fire(r):
                pltpu.async_copy(table_ref.at[kb.at[r]], cnt.at[r], sem)

            @pl.loop(0, qb)
            def _sel(r):
                pltpu.make_async_copy(
                    table_ref.at[kb.at[r]], cnt.at[r], sem).wait()
                for u in range(UB):
                    s = qoff + r * BLK + u * L
                    hit = cnt[r, pl.ds(u * L, L)] == gen
                    kcur[pl.ds(s, L)] = jnp.where(
                        hit, knbuf[pl.ds(s, L)], kcur[pl.ds(s, L)])

    pltpu.sync_copy(j_hbm.at[pl.ds(base, chunk)], jbuf)
    lane = lax.iota(jnp.int32, L)
    zeros = jnp.zeros((L,), jnp.float32)
    lane0 = jnp.where(lane == 0, jnp.float32(1.0), jnp.float32(0.0))
    signv = jnp.full((L,), jnp.float32(sign))
    acc31[pl.ds(0, L)] = zeros
    acc31[pl.ds(L - 1, L)] = zeros

    def _fire(b):
        p = b & 1
        pltpu.async_copy(z_hbm.at[ibuf.at[pl.ds(b * BLK, BLK)]], zi.at[p], sem)
        pltpu.async_copy(z_hbm.at[jbuf.at[pl.ds(b * BLK, BLK)]], zj.at[p], sem)
        pltpu.async_copy(z_hbm.at[kcur.at[pl.ds(b * BLK, BLK)]], zk.at[p], sem)

    _fire(0)

    def _block(b, vacc):
        pl.when(b + 1 < nb)(lambda: _fire(b + 1))
        p = b & 1
        pltpu.make_async_copy(
            z_hbm.at[ibuf.at[pl.ds(b * BLK, BLK)]], zi.at[p], sem).wait()
        pltpu.make_async_copy(
            z_hbm.at[jbuf.at[pl.ds(b * BLK, BLK)]], zj.at[p], sem).wait()
        pltpu.make_async_copy(
            z_hbm.at[kcur.at[pl.ds(b * BLK, BLK)]], zk.at[p], sem).wait()

        def _edge(e, acc_v):
            acc = jnp.zeros((L,), jnp.float32)
            for u in range(128 // L):
                a = zi[p, e, pl.ds(u * L, L)]
                pj = zj[p, e, pl.ds(u * L, L)]
                pk = zk[p, e, pl.ds(u * L, L)]
                d1 = a - pj
                d2 = a - pk
                acc = acc + (d1 * d1 - d2 * d2)
            v = acc
            for sh in (8, 4, 2, 1):
                acc31[pl.ds(0, L)] = v
                v = v + acc31[pl.ds(sh, L)]
            return acc_v + jnp.maximum(v * signv, 0.0) * lane0

        return lax.fori_loop(0, BLK, _edge, vacc, unroll=2)

    vacc = lax.fori_loop(0, nb, _block, jnp.zeros((L,), jnp.float32))
    vres[...] = vacc
    pltpu.sync_copy(vres, out_hbm.at[pl.ds(w * L, L)])


def _make_calls(n_nodes, n_edges, chunk):
    mesh = _mesh()
    insert = {}
    query = {}
    for gen, sign in ((1, 1.0), (2, -1.0)):
        insert[gen] = pl.kernel(
            functools.partial(_insert_body, n_nodes, chunk, gen),
            out_type=(),
            mesh=mesh,
            scratch_types=[
                pltpu.VMEM((chunk,), jnp.int32),
                pltpu.VMEM((chunk,), jnp.int32),
                pltpu.VMEM((chunk // IBLK + 1, IBLK), jnp.int32),
                pltpu.VMEM((IBLK,), jnp.int32),
                pltpu.SemaphoreType.DMA,
            ],
        )
        query[gen] = pl.kernel(
            functools.partial(_query_body, n_nodes, chunk, gen, sign),
            out_type=jax.ShapeDtypeStruct((NW * L,), jnp.float32),
            mesh=mesh,
            scratch_types=[
                pltpu.VMEM((chunk,), jnp.int32),
                pltpu.VMEM((chunk,), jnp.int32),
                pltpu.VMEM((chunk,), jnp.int32),
                pltpu.VMEM((chunk,), jnp.int32),
                pltpu.VMEM((chunk // BLK // 5, BLK), jnp.int32),
                pltpu.VMEM((chunk // BLK // 5, BLK), jnp.int32),
                pltpu.VMEM((2, BLK, 128), jnp.float32),
                pltpu.VMEM((2, BLK, 128), jnp.float32),
                pltpu.VMEM((2, BLK, 128), jnp.float32),
                pltpu.VMEM((2 * L - 1,), jnp.float32),
                pltpu.VMEM((L,), jnp.float32),
                pltpu.SemaphoreType.DMA,
            ],
        )
    return insert, query


def kernel(z, pos_edges, neg_edges):
    n_nodes, d_feat = z.shape
    n_edges = pos_edges.shape[1]
    assert d_feat == 128 and n_edges % NW == 0
    chunk = n_edges // NW
    assert chunk % BLK == 0

    key = jax.random.key(42)
    edge_sets = []
    for c, edges in ((1, pos_edges), (2, neg_edges)):
        sk = jax.random.fold_in(key, c)
        i = edges[0].astype(jnp.int32)
        j = edges[1].astype(jnp.int32)
        draws = [
            jax.random.randint(jax.random.fold_in(sk, t), (n_edges,), 0,
                               n_nodes, dtype=jnp.int32)
            for t in range(4)
        ]
        edge_sets.append((i, j, *draws))

    insert, query = _make_calls(n_nodes, n_edges, chunk)
    table = jax.new_ref(jnp.zeros((n_nodes * n_nodes + 8,), jnp.int32))

    total = jnp.float32(0.0)
    for gen, (i, j, k0, kn1, kn2, kn3) in ((1, edge_sets[0]),
                                           (2, edge_sets[1])):
        insert[gen](i, j, table)
        psum = query[gen](i, j, k0, kn1, kn2, kn3, z, table)
        total = total + jnp.sum(psum) / n_edges
    return total


# speedup vs baseline: 1.6254x; 1.6254x over previous
import functools

import jax
import jax.numpy as jnp
from jax import lax
from jax.experimental import pallas as pl
from jax.experimental.pallas import tpu as pltpu
from jax.experimental.pallas import tpu_sc as plsc

L = 16
NC = 2
NS = 16
NW = NC * NS
BLK = 80
UB = BLK // L


def _mesh():
    return plsc.VectorSubcoreMesh(core_axis_name="c", subcore_axis_name="s")


def _wid():
    return lax.axis_index("s") * NC + lax.axis_index("c")


IBLK = 80


def _insert_body(n_nodes, chunk, gen, i_hbm, j_hbm, table_ref,
                 ibuf, jbuf, kb, ones, sem):
    nfull = chunk // IBLK
    tail = chunk - nfull * IBLK
    nrows = nfull + (1 if tail else 0)
    pad_key = n_nodes * n_nodes
    w = _wid()
    base = w * chunk
    pltpu.sync_copy(i_hbm.at[pl.ds(base, chunk)], ibuf)
    pltpu.sync_copy(j_hbm.at[pl.ds(base, chunk)], jbuf)
    for u in range(IBLK // L):
        ones[pl.ds(u * L, L)] = jnp.full((L,), gen, jnp.int32)

    @pl.loop(0, nfull)
    def _keys(b):
        for u in range(IBLK // L):
            s = b * IBLK + u * L
            iv = ibuf[pl.ds(s, L)]
            jv = jbuf[pl.ds(s, L)]
            kb[b, pl.ds(u * L, L)] = iv * n_nodes + jv

    if tail:
        assert tail % L == 0
        for u in range(IBLK // L):
            s = nfull * IBLK + u * L
            if u * L < tail:
                iv = ibuf[pl.ds(s, L)]
                jv = jbuf[pl.ds(s, L)]
                kb[nfull, pl.ds(u * L, L)] = iv * n_nodes + jv
            else:
                kb[nfull, pl.ds(u * L, L)] = jnp.full((L,), pad_key, jnp.int32)

    @pl.loop(0, nrows)
    def _fire(b):
        pltpu.async_copy(ones, table_ref.at[kb.at[b]], sem)

    @pl.loop(0, nrows)
    def _drain(b):
        pltpu.make_async_copy(ones, table_ref.at[kb.at[b]], sem).wait()


def _query_body(n_nodes, chunk, gen, sign,
                i_hbm, j_hbm, k0_hbm, kn1_hbm, kn2_hbm, kn3_hbm,
                z_hbm, table_ref, out_hbm,
                ibuf, jbuf, kcur, knbuf, hitbuf, kb, cnt, zi, zj, zk, acc31,
                vres, sem):
    nb = chunk // BLK
    w = _wid()
    base = w * chunk
    pltpu.sync_copy(i_hbm.at[pl.ds(base, chunk)], ibuf)
    pltpu.sync_copy(k0_hbm.at[pl.ds(base, chunk)], kcur)

    nq = 5
    qb = nb // nq
    for t, kn_hbm in enumerate((kn1_hbm, kn2_hbm, kn3_hbm)):
        pltpu.sync_copy(kn_hbm.at[pl.ds(base, chunk)], knbuf)
        for q in range(nq):
            qoff = q * qb * BLK

            @pl.loop(0, qb)
            def _keys(r):
                for u in range(UB):
                    s = qoff + r * BLK + u * L
                    iv = ibuf[pl.ds(s, L)]
                    kv = kcur[pl.ds(s, L)]
                    key = iv * n_nodes + kv
                    if t > 0:
                        live = hitbuf[pl.ds(s, L)] == 1
                        key = jnp.where(live, key, jnp.full((L,), -1,
                                                            jnp.int32))
                    kb[r, pl.ds(u * L, L)] = key

            if t == 0:
                @pl.loop(0, qb)
                def _fire(r):
                    pltpu.async_copy(table_ref.at[kb.at[r]], cnt.at[r], sem)
            else:
                @pl.loop(0, qb)
                def _fire(r):
                    pltpu.async_copy(
                        table_ref.at[plsc.Indices(kb.at[r], -1)],
                        cnt.at[r], sem)

            @pl.loop(0, qb)
            def _sel(r):
                if t == 0:
                    pltpu.make_async_copy(
                        table_ref.at[kb.at[r]], cnt.at[r], sem).wait()
                else:
                    pltpu.make_async_copy(
                        table_ref.at[plsc.Indices(kb.at[r], -1)],
                        cnt.at[r], sem).wait()
                for u in range(UB):
                    s = qoff + r * BLK + u * L
                    hit = cnt[r, pl.ds(u * L, L)] == gen
                    if t > 0:
                        hit = hit & (hitbuf[pl.ds(s, L)] == 1)
                    hitbuf[pl.ds(s, L)] = jnp.where(
                        hit, jnp.full((L,), 1, jnp.int32),
                        jnp.zeros((L,), jnp.int32))
                    kcur[pl.ds(s, L)] = jnp.where(
                        hit, knbuf[pl.ds(s, L)], kcur[pl.ds(s, L)])

    pltpu.sync_copy(j_hbm.at[pl.ds(base, chunk)], jbuf)
    lane = lax.iota(jnp.int32, L)
    zeros = jnp.zeros((L,), jnp.float32)
    lane0 = jnp.where(lane == 0, jnp.float32(1.0), jnp.float32(0.0))
    signv = jnp.full((L,), jnp.float32(sign))
    acc31[pl.ds(0, L)] = zeros
    acc31[pl.ds(L - 1, L)] = zeros

    def _fire(b):
        p = b & 1
        pltpu.async_copy(z_hbm.at[ibuf.at[pl.ds(b * BLK, BLK)]], zi.at[p], sem)
        pltpu.async_copy(z_hbm.at[jbuf.at[pl.ds(b * BLK, BLK)]], zj.at[p], sem)
        pltpu.async_copy(z_hbm.at[kcur.at[pl.ds(b * BLK, BLK)]], zk.at[p], sem)

    _fire(0)

    def _block(b, vacc):
        pl.when(b + 1 < nb)(lambda: _fire(b + 1))
        p = b & 1
        pltpu.make_async_copy(
            z_hbm.at[ibuf.at[pl.ds(b * BLK, BLK)]], zi.at[p], sem).wait()
        pltpu.make_async_copy(
            z_hbm.at[jbuf.at[pl.ds(b * BLK, BLK)]], zj.at[p], sem).wait()
        pltpu.make_async_copy(
            z_hbm.at[kcur.at[pl.ds(b * BLK, BLK)]], zk.at[p], sem).wait()

        def _edge(e, acc_v):
            acc = jnp.zeros((L,), jnp.float32)
            for u in range(128 // L):
                a = zi[p, e, pl.ds(u * L, L)]
                pj = zj[p, e, pl.ds(u * L, L)]
                pk = zk[p, e, pl.ds(u * L, L)]
                d1 = a - pj
                d2 = a - pk
                acc = acc + (d1 * d1 - d2 * d2)
            v = acc
            for sh in (8, 4, 2, 1):
                acc31[pl.ds(0, L)] = v
                v = v + acc31[pl.ds(sh, L)]
            return acc_v + jnp.maximum(v * signv, 0.0) * lane0

        return lax.fori_loop(0, BLK, _edge, vacc, unroll=2)

    vacc = lax.fori_loop(0, nb, _block, jnp.zeros((L,), jnp.float32))
    vres[...] = vacc
    pltpu.sync_copy(vres, out_hbm.at[pl.ds(w * L, L)])


def _make_calls(n_nodes, n_edges, chunk):
    mesh = _mesh()
    insert = {}
    query = {}
    for gen, sign in ((1, 1.0), (2, -1.0)):
        insert[gen] = pl.kernel(
            functools.partial(_insert_body, n_nodes, chunk, gen),
            out_type=(),
            mesh=mesh,
            scratch_types=[
                pltpu.VMEM((chunk,), jnp.int32),
                pltpu.VMEM((chunk,), jnp.int32),
                pltpu.VMEM((chunk // IBLK + 1, IBLK), jnp.int32),
                pltpu.VMEM((IBLK,), jnp.int32),
                pltpu.SemaphoreType.DMA,
            ],
        )
        query[gen] = pl.kernel(
            functools.partial(_query_body, n_nodes, chunk, gen, sign),
            out_type=jax.ShapeDtypeStruct((NW * L,), jnp.float32),
            mesh=mesh,
            scratch_types=[
                pltpu.VMEM((chunk,), jnp.int32),
                pltpu.VMEM((chunk,), jnp.int32),
                pltpu.VMEM((chunk,), jnp.int32),
                pltpu.VMEM((chunk,), jnp.int32),
                pltpu.VMEM((chunk,), jnp.int32),
                pltpu.VMEM((chunk // BLK // 5, BLK), jnp.int32),
                pltpu.VMEM((chunk // BLK // 5, BLK), jnp.int32),
                pltpu.VMEM((2, BLK, 128), jnp.float32),
                pltpu.VMEM((2, BLK, 128), jnp.float32),
                pltpu.VMEM((2, BLK, 128), jnp.float32),
                pltpu.VMEM((2 * L - 1,), jnp.float32),
                pltpu.VMEM((L,), jnp.float32),
                pltpu.SemaphoreType.DMA,
            ],
        )
    return insert, query


def kernel(z, pos_edges, neg_edges):
    n_nodes, d_feat = z.shape
    n_edges = pos_edges.shape[1]
    assert d_feat == 128 and n_edges % NW == 0
    chunk = n_edges // NW
    assert chunk % BLK == 0

    key = jax.random.key(42)
    edge_sets = []
    for c, edges in ((1, pos_edges), (2, neg_edges)):
        sk = jax.random.fold_in(key, c)
        i = edges[0].astype(jnp.int32)
        j = edges[1].astype(jnp.int32)
        draws = [
            jax.random.randint(jax.random.fold_in(sk, t), (n_edges,), 0,
                               n_nodes, dtype=jnp.int32)
            for t in range(4)
        ]
        edge_sets.append((i, j, *draws))

    insert, query = _make_calls(n_nodes, n_edges, chunk)
    table = jax.new_ref(jnp.zeros((n_nodes * n_nodes + 8,), jnp.int32))

    total = jnp.float32(0.0)
    for gen, (i, j, k0, kn1, kn2, kn3) in ((1, edge_sets[0]),
                                           (2, edge_sets[1])):
        insert[gen](i, j, table)
        psum = query[gen](i, j, k0, kn1, kn2, kn3, z, table)
        total = total + jnp.sum(psum) / n_edges
    return total
